# baseline (speedup 1.0000x reference)
import functools

import jax
import jax.numpy as jnp
import numpy as np
from jax.experimental import pallas as pl

N_PCT = 10
MIN_PCT = 5
MAX_PCT = 95


def _pct_constants(n):
    fracs = np.linspace(MIN_PCT / 100.0, MAX_PCT / 100.0, N_PCT)
    idx_float = fracs * (n - 1)
    idx_lower = np.floor(idx_float).astype(np.int32)
    idx_upper = np.ceil(idx_float).astype(np.int32)
    w_upper = (idx_float - idx_lower).astype(np.float32)
    return idx_lower, idx_upper, w_upper


def _bitrev(r, nbits):
    return int(format(r, "0{}b".format(nbits))[::-1], 2)


def _body(x_ref, o_ref, *, n):
    dc = x_ref.shape[2]
    logn = int(np.log2(n))
    u = jax.lax.bitcast_convert_type(x_ref[0], jnp.int32)
    v = u ^ ((u >> 31) & 0x7FFFFFFF)
    row3 = jax.lax.broadcasted_iota(jnp.int32, (n // 8, 8, 1), 1)
    bit0_3 = [((row3 >> p) & 1) == 0 for p in range(3)]
    row = jax.lax.broadcasted_iota(jnp.int32, (n, 1), 0)
    mbit = [(row >> p) & 1 for p in range(logn)]
    v = v ^ -mbit[logn - 2]
    G = 4
    for k in range(1, logn + 1):
        ps = list(range(logn - k, logn))
        for p in [q for q in ps if q < 3]:
            dist = 1 << p
            w = v.reshape(n // 8, 8, dc)
            up = jnp.roll(w, -dist, axis=1)
            down = jnp.roll(w, dist, axis=1)
            v = jnp.where(
                bit0_3[p], jnp.minimum(w, up), jnp.maximum(w, down)
            ).reshape(n, dc)
        big = [q for q in ps if q >= 3]
        pos = 0
        while pos < len(big):
            group = big[pos : pos + G]
            pos += len(group)
            g = len(group)
            p0 = group[0]
            dist = 1 << p0
            m = n // ((1 << g) * dist)
            w = v.reshape(m, 1 << g, dist, dc)
            sl = [w[:, q] for q in range(1 << g)]
            for e in range(g):
                step = 1 << e
                for q in range(1 << g):
                    if q & step:
                        continue
                    a, b_ = sl[q], sl[q ^ step]
                    sl[q] = jnp.minimum(a, b_)
                    sl[q ^ step] = jnp.maximum(a, b_)
            v = jnp.stack(sl, axis=1).reshape(n, dc)
        if k < logn - 1:
            v = v ^ -(mbit[logn - 1 - k] ^ mbit[logn - 2 - k])
        elif k == logn - 1:
            v = v ^ -mbit[0]

    idx_lower, idx_upper, w_upper = _pct_constants(n)
    rows = []
    for p in range(N_PCT):
        rl = _bitrev(int(idx_lower[p]), logn)
        ru = _bitrev(int(idx_upper[p]), logn)
        rows.append(v[rl : rl + 1, :])
        rows.append(v[ru : ru + 1, :])
    keys = jnp.concatenate(rows, axis=0)
    ui = keys ^ ((keys >> 31) & 0x7FFFFFFF)
    f = jax.lax.bitcast_convert_type(ui, jnp.float32)
    out = [
        f[2 * p : 2 * p + 1] * (1.0 - float(w_upper[p]))
        + f[2 * p + 1 : 2 * p + 2] * float(w_upper[p])
        for p in range(N_PCT)
    ]
    o_ref[0] = jnp.concatenate(out, axis=0)


@jax.jit
def kernel(x):
    b, n, d = x.shape
    DC = 256
    out = pl.pallas_call(
        functools.partial(_body, n=n),
        grid=(b, d // DC),
        in_specs=[pl.BlockSpec((1, n, DC), lambda i, j: (i, 0, j))],
        out_specs=pl.BlockSpec((1, N_PCT, DC), lambda i, j: (i, 0, j)),
        out_shape=jax.ShapeDtypeStruct((b, N_PCT, d), jnp.float32),
    )(x)
    return jnp.transpose(out, (0, 2, 1)).reshape(b, d * N_PCT)


# speedup vs baseline: 1.1622x; 1.1622x over previous
import functools

import jax
import jax.numpy as jnp
import numpy as np
from jax.experimental import pallas as pl

N_PCT = 10
MIN_PCT = 5
MAX_PCT = 95


def _pct_constants(n):
    fracs = np.linspace(MIN_PCT / 100.0, MAX_PCT / 100.0, N_PCT)
    idx_float = fracs * (n - 1)
    idx_lower = np.floor(idx_float).astype(np.int32)
    idx_upper = np.ceil(idx_float).astype(np.int32)
    w_upper = (idx_float - idx_lower).astype(np.float32)
    return idx_lower, idx_upper, w_upper


def _bitrev(r, nbits):
    return int(format(r, "0{}b".format(nbits))[::-1], 2)


def _body(x_ref, o_ref, *, n):
    dc = x_ref.shape[2]
    logn = int(np.log2(n))
    u = jax.lax.bitcast_convert_type(x_ref[0], jnp.int32)
    v = u ^ ((u >> 31) & 0x7FFFFFFF)
    row3 = jax.lax.broadcasted_iota(jnp.int32, (n // 8, 8, 1), 1)
    bit0_3 = [((row3 >> p) & 1) == 0 for p in range(3)]
    row = jax.lax.broadcasted_iota(jnp.int32, (n, 1), 0)
    mbit = [(row >> p) & 1 for p in range(logn)]
    v = v ^ -mbit[logn - 2]
    G = 5
    for k in range(1, logn + 1):
        ps = list(range(logn - k, logn))
        for p in [q for q in ps if q < 3]:
            dist = 1 << p
            w = v.reshape(n // 8, 8, dc)
            up = jnp.roll(w, -dist, axis=1)
            down = jnp.roll(w, dist, axis=1)
            v = jnp.where(
                bit0_3[p], jnp.minimum(w, up), jnp.maximum(w, down)
            ).reshape(n, dc)
        big = [q for q in ps if q >= 3]
        pos = 0
        while pos < len(big):
            group = big[pos : pos + G]
            pos += len(group)
            g = len(group)
            p0 = group[0]
            dist = 1 << p0
            m = n // ((1 << g) * dist)
            w = v.reshape(m, 1 << g, dist, dc)
            sl = [w[:, q] for q in range(1 << g)]
            for e in range(g):
                step = 1 << e
                for q in range(1 << g):
                    if q & step:
                        continue
                    a, b_ = sl[q], sl[q ^ step]
                    sl[q] = jnp.minimum(a, b_)
                    sl[q ^ step] = jnp.maximum(a, b_)
            v = jnp.stack(sl, axis=1).reshape(n, dc)
        if k < logn - 1:
            v = v ^ -(mbit[logn - 1 - k] ^ mbit[logn - 2 - k])
        elif k == logn - 1:
            v = v ^ -mbit[0]

    idx_lower, idx_upper, w_upper = _pct_constants(n)
    rows = []
    for p in range(N_PCT):
        rl = _bitrev(int(idx_lower[p]), logn)
        ru = _bitrev(int(idx_upper[p]), logn)
        rows.append(v[rl : rl + 1, :])
        rows.append(v[ru : ru + 1, :])
    keys = jnp.concatenate(rows, axis=0)
    ui = keys ^ ((keys >> 31) & 0x7FFFFFFF)
    f = jax.lax.bitcast_convert_type(ui, jnp.float32)
    out = [
        f[2 * p : 2 * p + 1] * (1.0 - float(w_upper[p]))
        + f[2 * p + 1 : 2 * p + 2] * float(w_upper[p])
        for p in range(N_PCT)
    ]
    o_ref[0] = jnp.concatenate(out, axis=0)


@jax.jit
def kernel(x):
    b, n, d = x.shape
    DC = 128
    out = pl.pallas_call(
        functools.partial(_body, n=n),
        grid=(b, d // DC),
        in_specs=[pl.BlockSpec((1, n, DC), lambda i, j: (i, 0, j))],
        out_specs=pl.BlockSpec((1, N_PCT, DC), lambda i, j: (i, 0, j)),
        out_shape=jax.ShapeDtypeStruct((b, N_PCT, d), jnp.float32),
    )(x)
    return jnp.transpose(out, (0, 2, 1)).reshape(b, d * N_PCT)


# speedup vs baseline: 1.1625x; 1.0003x over previous
import functools

import jax
import jax.numpy as jnp
import numpy as np
from jax.experimental import pallas as pl

N_PCT = 10
MIN_PCT = 5
MAX_PCT = 95


def _pct_constants(n):
    fracs = np.linspace(MIN_PCT / 100.0, MAX_PCT / 100.0, N_PCT)
    idx_float = fracs * (n - 1)
    idx_lower = np.floor(idx_float).astype(np.int32)
    idx_upper = np.ceil(idx_float).astype(np.int32)
    w_upper = (idx_float - idx_lower).astype(np.float32)
    return idx_lower, idx_upper, w_upper


def _bitrev(r, nbits):
    return int(format(r, "0{}b".format(nbits))[::-1], 2)


def _body(x_ref, o_ref, *, n):
    dc = x_ref.shape[2]
    logn = int(np.log2(n))
    u = jax.lax.bitcast_convert_type(x_ref[0], jnp.int32)
    v = u ^ ((u >> 31) & 0x7FFFFFFF)
    row3 = jax.lax.broadcasted_iota(jnp.int32, (n // 8, 8, 1), 1)
    bit0_3 = [((row3 >> p) & 1) == 0 for p in range(3)]
    row = jax.lax.broadcasted_iota(jnp.int32, (n, 1), 0)
    mbit = [(row >> p) & 1 for p in range(logn)]
    v = v ^ -mbit[logn - 2]
    G = 5
    for k in range(1, logn + 1):
        ps = list(range(logn - k, logn))
        for p in [q for q in ps if q < 3]:
            dist = 1 << p
            w = v.reshape(n // 8, 8, dc)
            up = jnp.roll(w, -dist, axis=1)
            down = jnp.roll(w, dist, axis=1)
            v = jnp.where(
                bit0_3[p], jnp.minimum(w, up), jnp.maximum(w, down)
            ).reshape(n, dc)
        big = [q for q in ps if q >= 3]
        pos = 0
        while pos < len(big):
            group = big[pos : pos + G]
            pos += len(group)
            g = len(group)
            p0 = group[0]
            dist = 1 << p0
            m = n // ((1 << g) * dist)
            w = v.reshape(m, 1 << g, dist, dc)
            sl = [w[:, q] for q in range(1 << g)]
            for e in range(g):
                step = 1 << e
                for q in range(1 << g):
                    if q & step:
                        continue
                    a, b_ = sl[q], sl[q ^ step]
                    sl[q] = jnp.minimum(a, b_)
                    sl[q ^ step] = jnp.maximum(a, b_)
            if k == logn and pos == len(big):
                final_sl, final_p0, final_g = sl, p0, g
            else:
                v = jnp.stack(sl, axis=1).reshape(n, dc)
        if k < logn - 1:
            v = v ^ -(mbit[logn - 1 - k] ^ mbit[logn - 2 - k])
        elif k == logn - 1:
            v = v ^ -mbit[0]

    idx_lower, idx_upper, w_upper = _pct_constants(n)

    def _take(r):
        q = (r >> final_p0) & ((1 << final_g) - 1)
        mi = r >> (final_p0 + final_g)
        t = r & ((1 << final_p0) - 1)
        return final_sl[q][mi, t : t + 1, :]

    rows = []
    for p in range(N_PCT):
        rows.append(_take(_bitrev(int(idx_lower[p]), logn)))
        rows.append(_take(_bitrev(int(idx_upper[p]), logn)))
    keys = jnp.concatenate(rows, axis=0)
    ui = keys ^ ((keys >> 31) & 0x7FFFFFFF)
    f = jax.lax.bitcast_convert_type(ui, jnp.float32)
    out = [
        f[2 * p : 2 * p + 1] * (1.0 - float(w_upper[p]))
        + f[2 * p + 1 : 2 * p + 2] * float(w_upper[p])
        for p in range(N_PCT)
    ]
    o_ref[0] = jnp.concatenate(out, axis=0)


@jax.jit
def kernel(x):
    b, n, d = x.shape
    DC = 128
    out = pl.pallas_call(
        functools.partial(_body, n=n),
        grid=(b, d // DC),
        in_specs=[pl.BlockSpec((1, n, DC), lambda i, j: (i, 0, j))],
        out_specs=pl.BlockSpec((1, N_PCT, DC), lambda i, j: (i, 0, j)),
        out_shape=jax.ShapeDtypeStruct((b, N_PCT, d), jnp.float32),
    )(x)
    return jnp.transpose(out, (0, 2, 1)).reshape(b, d * N_PCT)


# speedup vs baseline: 1.2537x; 1.0785x over previous
import functools

import jax
import jax.numpy as jnp
import numpy as np
from jax.experimental import pallas as pl

N_PCT = 10
MIN_PCT = 5
MAX_PCT = 95


def _pct_constants(n):
    fracs = np.linspace(MIN_PCT / 100.0, MAX_PCT / 100.0, N_PCT)
    idx_float = fracs * (n - 1)
    idx_lower = np.floor(idx_float).astype(np.int32)
    idx_upper = np.ceil(idx_float).astype(np.int32)
    w_upper = (idx_float - idx_lower).astype(np.float32)
    return idx_lower, idx_upper, w_upper


def _bitrev(r, nbits):
    return int(format(r, "0{}b".format(nbits))[::-1], 2)


def _body(x_ref, o_ref, *, n):
    dc = x_ref.shape[2]
    logn = int(np.log2(n))
    v = x_ref[0]
    row3 = jax.lax.broadcasted_iota(jnp.int32, (n // 8, 8, 1), 1)
    bit0_3 = [((row3 >> p) & 1) == 0 for p in range(3)]
    row = jax.lax.broadcasted_iota(jnp.int32, (n, 1), 0)
    mbit = [(row >> p) & 1 for p in range(logn)]

    def _sgn(bit):
        return jnp.where(bit == 0, 1.0, -1.0).astype(jnp.float32)

    v = v * _sgn(mbit[logn - 2])
    G = 5
    for k in range(1, logn + 1):
        ps = list(range(logn - k, logn))
        for p in [q for q in ps if q < 3]:
            dist = 1 << p
            w = v.reshape(n // 8, 8, dc)
            up = jnp.roll(w, -dist, axis=1)
            down = jnp.roll(w, dist, axis=1)
            v = jnp.where(
                bit0_3[p], jnp.minimum(w, up), jnp.maximum(w, down)
            ).reshape(n, dc)
        big = [q for q in ps if q >= 3]
        pos = 0
        while pos < len(big):
            group = big[pos : pos + G]
            pos += len(group)
            g = len(group)
            p0 = group[0]
            dist = 1 << p0
            m = n // ((1 << g) * dist)
            w = v.reshape(m, 1 << g, dist, dc)
            sl = [w[:, q] for q in range(1 << g)]
            for e in range(g):
                step = 1 << e
                for q in range(1 << g):
                    if q & step:
                        continue
                    a, b_ = sl[q], sl[q ^ step]
                    sl[q] = jnp.minimum(a, b_)
                    sl[q ^ step] = jnp.maximum(a, b_)
            if k == logn and pos == len(big):
                final_sl, final_p0, final_g = sl, p0, g
            else:
                v = jnp.stack(sl, axis=1).reshape(n, dc)
        if k < logn - 1:
            v = v * _sgn(mbit[logn - 1 - k] ^ mbit[logn - 2 - k])
        elif k == logn - 1:
            v = v * _sgn(mbit[0])

    idx_lower, idx_upper, w_upper = _pct_constants(n)

    def _take(r):
        q = (r >> final_p0) & ((1 << final_g) - 1)
        mi = r >> (final_p0 + final_g)
        t = r & ((1 << final_p0) - 1)
        return final_sl[q][mi, t : t + 1, :]

    rows = []
    for p in range(N_PCT):
        rows.append(_take(_bitrev(int(idx_lower[p]), logn)))
        rows.append(_take(_bitrev(int(idx_upper[p]), logn)))
    f = jnp.concatenate(rows, axis=0)
    out = [
        f[2 * p : 2 * p + 1] * (1.0 - float(w_upper[p]))
        + f[2 * p + 1 : 2 * p + 2] * float(w_upper[p])
        for p in range(N_PCT)
    ]
    o_ref[0] = jnp.concatenate(out, axis=0)


@jax.jit
def kernel(x):
    b, n, d = x.shape
    DC = 128
    out = pl.pallas_call(
        functools.partial(_body, n=n),
        grid=(b, d // DC),
        in_specs=[pl.BlockSpec((1, n, DC), lambda i, j: (i, 0, j))],
        out_specs=pl.BlockSpec((1, N_PCT, DC), lambda i, j: (i, 0, j)),
        out_shape=jax.ShapeDtypeStruct((b, N_PCT, d), jnp.float32),
    )(x)
    return jnp.transpose(out, (0, 2, 1)).reshape(b, d * N_PCT)


# speedup vs baseline: 1.2728x; 1.0152x over previous
import functools

import jax
import jax.numpy as jnp
import numpy as np
from jax.experimental import pallas as pl

N_PCT = 10
MIN_PCT = 5
MAX_PCT = 95


def _pct_constants(n):
    fracs = np.linspace(MIN_PCT / 100.0, MAX_PCT / 100.0, N_PCT)
    idx_float = fracs * (n - 1)
    idx_lower = np.floor(idx_float).astype(np.int32)
    idx_upper = np.ceil(idx_float).astype(np.int32)
    w_upper = (idx_float - idx_lower).astype(np.float32)
    return idx_lower, idx_upper, w_upper


def _bitrev(r, nbits):
    return int(format(r, "0{}b".format(nbits))[::-1], 2)


def _body(x_ref, o_ref, *, n):
    dc = x_ref.shape[2]
    logn = int(np.log2(n))
    v = x_ref[0]
    row3 = jax.lax.broadcasted_iota(jnp.int32, (n // 8, 8, 1), 1)
    bit0_3 = [((row3 >> p) & 1) == 0 for p in range(3)]
    row = jax.lax.broadcasted_iota(jnp.int32, (n, 1), 0)
    mbit = [(row >> p) & 1 for p in range(logn)]

    def _sgn(bit):
        return jnp.where(bit == 0, 1.0, -1.0).astype(jnp.float32)

    v = v * _sgn(mbit[logn - 2])
    G = 5
    for k in range(1, logn + 1):
        ps = list(range(logn - k, logn))
        for p in [q for q in ps if q < 3]:
            dist = 1 << p
            up = jnp.roll(v, -dist, axis=0)
            down = jnp.roll(v, dist, axis=0)
            v = jnp.where(
                bit0_3[p].reshape(n, 1),
                jnp.minimum(v, up),
                jnp.maximum(v, down),
            )
        big = [q for q in ps if q >= 3]
        pos = 0
        while pos < len(big):
            group = big[pos : pos + G]
            pos += len(group)
            g = len(group)
            p0 = group[0]
            dist = 1 << p0
            m = n // ((1 << g) * dist)
            w = v.reshape(m, 1 << g, dist, dc)
            sl = [w[:, q] for q in range(1 << g)]
            for e in range(g):
                step = 1 << e
                for q in range(1 << g):
                    if q & step:
                        continue
                    a, b_ = sl[q], sl[q ^ step]
                    sl[q] = jnp.minimum(a, b_)
                    sl[q ^ step] = jnp.maximum(a, b_)
            if k == logn and pos == len(big):
                final_sl, final_p0, final_g = sl, p0, g
            else:
                v = jnp.stack(sl, axis=1).reshape(n, dc)
        if k < logn - 1:
            v = v * _sgn(mbit[logn - 1 - k] ^ mbit[logn - 2 - k])
        elif k == logn - 1:
            v = v * _sgn(mbit[0])

    idx_lower, idx_upper, w_upper = _pct_constants(n)

    def _take(r):
        q = (r >> final_p0) & ((1 << final_g) - 1)
        mi = r >> (final_p0 + final_g)
        t = r & ((1 << final_p0) - 1)
        return final_sl[q][mi, t : t + 1, :]

    rows = []
    for p in range(N_PCT):
        rows.append(_take(_bitrev(int(idx_lower[p]), logn)))
        rows.append(_take(_bitrev(int(idx_upper[p]), logn)))
    f = jnp.concatenate(rows, axis=0)
    out = [
        f[2 * p : 2 * p + 1] * (1.0 - float(w_upper[p]))
        + f[2 * p + 1 : 2 * p + 2] * float(w_upper[p])
        for p in range(N_PCT)
    ]
    o_ref[0] = jnp.concatenate(out, axis=0)


@jax.jit
def kernel(x):
    b, n, d = x.shape
    DC = 128
    out = pl.pallas_call(
        functools.partial(_body, n=n),
        grid=(b, d // DC),
        in_specs=[pl.BlockSpec((1, n, DC), lambda i, j: (i, 0, j))],
        out_specs=pl.BlockSpec((1, N_PCT, DC), lambda i, j: (i, 0, j)),
        out_shape=jax.ShapeDtypeStruct((b, N_PCT, d), jnp.float32),
    )(x)
    return jnp.transpose(out, (0, 2, 1)).reshape(b, d * N_PCT)


# speedup vs baseline: 1.2758x; 1.0024x over previous
import functools

import jax
import jax.numpy as jnp
import numpy as np
from jax.experimental import pallas as pl

N_PCT = 10
MIN_PCT = 5
MAX_PCT = 95


def _pct_constants(n):
    fracs = np.linspace(MIN_PCT / 100.0, MAX_PCT / 100.0, N_PCT)
    idx_float = fracs * (n - 1)
    idx_lower = np.floor(idx_float).astype(np.int32)
    idx_upper = np.ceil(idx_float).astype(np.int32)
    w_upper = (idx_float - idx_lower).astype(np.float32)
    return idx_lower, idx_upper, w_upper


def _bitrev(r, nbits):
    return int(format(r, "0{}b".format(nbits))[::-1], 2)


def _body(x_ref, o_ref, *, n):
    dc = x_ref.shape[2]
    logn = int(np.log2(n))
    v = x_ref[0]
    row3 = jax.lax.broadcasted_iota(jnp.int32, (n // 8, 8, 1), 1)
    bit0_3 = [((row3 >> p) & 1) == 0 for p in range(3)]
    row = jax.lax.broadcasted_iota(jnp.int32, (n, 1), 0)
    mbit = [(row >> p) & 1 for p in range(logn)]

    def _sgn(bit):
        return jnp.where(bit == 0, 1.0, -1.0).astype(jnp.float32)

    v = v * _sgn(mbit[logn - 2])
    G = 6
    for k in range(1, logn + 1):
        ps = list(range(logn - k, logn))
        for p in [q for q in ps if q < 3]:
            dist = 1 << p
            up = jnp.roll(v, -dist, axis=0)
            down = jnp.roll(v, dist, axis=0)
            v = jnp.where(
                bit0_3[p].reshape(n, 1),
                jnp.minimum(v, up),
                jnp.maximum(v, down),
            )
        big = [q for q in ps if q >= 3]
        pos = 0
        while pos < len(big):
            group = big[pos : pos + G]
            pos += len(group)
            g = len(group)
            p0 = group[0]
            dist = 1 << p0
            m = n // ((1 << g) * dist)
            w = v.reshape(m, 1 << g, dist, dc)
            sl = [w[:, q] for q in range(1 << g)]
            for e in range(g):
                step = 1 << e
                for q in range(1 << g):
                    if q & step:
                        continue
                    a, b_ = sl[q], sl[q ^ step]
                    sl[q] = jnp.minimum(a, b_)
                    sl[q ^ step] = jnp.maximum(a, b_)
            if k == logn and pos == len(big):
                final_sl, final_p0, final_g = sl, p0, g
            else:
                v = jnp.stack(sl, axis=1).reshape(n, dc)
        if k < logn - 1:
            v = v * _sgn(mbit[logn - 1 - k] ^ mbit[logn - 2 - k])
        elif k == logn - 1:
            v = v * _sgn(mbit[0])

    idx_lower, idx_upper, w_upper = _pct_constants(n)

    def _take(r):
        q = (r >> final_p0) & ((1 << final_g) - 1)
        mi = r >> (final_p0 + final_g)
        t = r & ((1 << final_p0) - 1)
        return final_sl[q][mi, t : t + 1, :]

    rows = []
    for p in range(N_PCT):
        rows.append(_take(_bitrev(int(idx_lower[p]), logn)))
        rows.append(_take(_bitrev(int(idx_upper[p]), logn)))
    f = jnp.concatenate(rows, axis=0)
    out = [
        f[2 * p : 2 * p + 1] * (1.0 - float(w_upper[p]))
        + f[2 * p + 1 : 2 * p + 2] * float(w_upper[p])
        for p in range(N_PCT)
    ]
    o_ref[0] = jnp.concatenate(out, axis=0)


@jax.jit
def kernel(x):
    b, n, d = x.shape
    DC = 128
    out = pl.pallas_call(
        functools.partial(_body, n=n),
        grid=(b, d // DC),
        in_specs=[pl.BlockSpec((1, n, DC), lambda i, j: (i, 0, j))],
        out_specs=pl.BlockSpec((1, N_PCT, DC), lambda i, j: (i, 0, j)),
        out_shape=jax.ShapeDtypeStruct((b, N_PCT, d), jnp.float32),
    )(x)
    return jnp.transpose(out, (0, 2, 1)).reshape(b, d * N_PCT)
